# baseline (speedup 1.0000x reference)
import functools

import jax
import jax.numpy as jnp
from jax import lax
from jax.experimental import pallas as pl
from jax.experimental.pallas import tpu as pltpu
from jax.experimental.pallas import tpu_sc as plsc

N = 10000
NP = 10240
E = 320000
D_IN = 128
H = 64
NSC = 2
NTILE = 16
NW = NSC * NTILE
E_PER_W = E // NW
CHUNK = 80
NCHUNK = E_PER_W // CHUNK
RPT = NP // NTILE
NBUF = 5
LAG = 4
NOUT = NCHUNK // NBUF

BM = 2000
NB = N // BM


def _mesh():
    return plsc.VectorSubcoreMesh(core_axis_name="c", subcore_axis_name="s")


@functools.partial(
    pl.kernel,
    out_type=(
        jax.ShapeDtypeStruct((NSC * NP, H), jnp.bfloat16),
        jax.ShapeDtypeStruct((NSC * NP,), jnp.float32),
    ),
    mesh=_mesh(),
    compiler_params=pltpu.CompilerParams(use_tc_tiling_on_sc=False),
    scratch_types=(
        pltpu.VMEM((E_PER_W,), jnp.int32),
        pltpu.VMEM((E_PER_W,), jnp.int32),
        pltpu.VMEM((NBUF, CHUNK, H), jnp.bfloat16),
        pltpu.VMEM((CHUNK,), jnp.float32),
        pltpu.VMEM_SHARED((NP, H), jnp.bfloat16),
        pltpu.VMEM_SHARED((NP,), jnp.float32),
        pltpu.SemaphoreType.DMA((NBUF,)),
        pltpu.SemaphoreType.DMA((NBUF,)),
        pltpu.SemaphoreType.DMA((NBUF,)),
    ),
)
def _seg_sum_count(y_hbm, srcf_hbm, dstf_hbm, zr_hbm, zc_hbm, ones_hbm,
                   acc_out, cnt_out,
                   src_v, dst_v, rows_v, ones_v, acc_sh, cnt_sh,
                   gsem, ssem, csem):
    c = lax.axis_index("c")
    s = lax.axis_index("s")
    wid = c * NTILE + s
    rbase = s * RPT
    pltpu.sync_copy(zr_hbm, acc_sh.at[pl.ds(rbase, RPT)])
    pltpu.sync_copy(zc_hbm, cnt_sh.at[pl.ds(rbase, RPT)])
    ebase = pl.multiple_of(wid * E_PER_W, 8)
    pltpu.sync_copy(srcf_hbm.at[pl.ds(ebase, E_PER_W)], src_v)
    pltpu.sync_copy(dstf_hbm.at[pl.ds(ebase, E_PER_W)], dst_v)
    pltpu.sync_copy(ones_hbm, ones_v)
    plsc.subcore_barrier()
    obase = c * NP + rbase

    def start_gather(j, b):
        pltpu.async_copy(y_hbm.at[src_v.at[pl.ds(j * CHUNK, CHUNK)]], rows_v.at[b], gsem.at[b])

    def wait_gather(j, b):
        pltpu.make_async_copy(y_hbm.at[src_v.at[pl.ds(j * CHUNK, CHUNK)]], rows_v.at[b],
                              gsem.at[b]).wait()

    def start_scatter(j, b):
        pltpu.async_copy(rows_v.at[b], acc_sh.at[dst_v.at[pl.ds(j * CHUNK, CHUNK)]], ssem.at[b],
                         add=True)

    def wait_scatter(j, b):
        pltpu.make_async_copy(rows_v.at[b], acc_sh.at[dst_v.at[pl.ds(j * CHUNK, CHUNK)]],
                              ssem.at[b]).wait()

    def start_count(j, b):
        pltpu.async_copy(ones_v, cnt_sh.at[dst_v.at[pl.ds(j * CHUNK, CHUNK)]], csem.at[b], add=True)

    def wait_count(j, b):
        pltpu.make_async_copy(ones_v, cnt_sh.at[dst_v.at[pl.ds(j * CHUNK, CHUNK)]],
                              csem.at[b]).wait()

    for b in range(LAG):
        start_gather(b, b)

    def group(g, carry):
        for b in range(NBUF):
            v = g * NBUF + b
            jg = v + LAG
            bb = (b + LAG) % NBUF

            @pl.when(jg < NCHUNK)
            def _():
                @pl.when(jg >= NBUF)
                def _():
                    wait_scatter(jg - NBUF, bb)
                start_gather(jg, bb)

            wait_gather(v, b)
            start_scatter(v, b)

            @pl.when(v >= NBUF)
            def _():
                wait_count(v - NBUF, b)
            start_count(v, b)
        return carry

    lax.fori_loop(0, NOUT, group, 0)
    for b in range(NBUF):
        wait_scatter(NCHUNK - NBUF + b, b)
        wait_count(NCHUNK - NBUF + b, b)
    plsc.subcore_barrier()

    pltpu.sync_copy(acc_sh.at[pl.ds(rbase, RPT)],
                    acc_out.at[pl.ds(obase, RPT)])
    pltpu.sync_copy(cnt_sh.at[pl.ds(rbase, RPT)],
                    cnt_out.at[pl.ds(obase, RPT)])


@functools.partial(
    pl.kernel,
    out_type=jax.ShapeDtypeStruct((NSC * NP, H), jnp.bfloat16),
    mesh=_mesh(),
    compiler_params=pltpu.CompilerParams(use_tc_tiling_on_sc=False),
    scratch_types=(
        pltpu.VMEM((E_PER_W,), jnp.int32),
        pltpu.VMEM((E_PER_W,), jnp.int32),
        pltpu.VMEM((NBUF, CHUNK, H), jnp.bfloat16),
        pltpu.VMEM_SHARED((NP, H), jnp.bfloat16),
        pltpu.SemaphoreType.DMA((NBUF,)),
        pltpu.SemaphoreType.DMA((NBUF,)),
    ),
)
def _seg_sum(y_hbm, srcf_hbm, dstf_hbm, zr_hbm,
             acc_out,
             src_v, dst_v, rows_v, acc_sh, gsem, ssem):
    c = lax.axis_index("c")
    s = lax.axis_index("s")
    wid = c * NTILE + s
    rbase = s * RPT
    pltpu.sync_copy(zr_hbm, acc_sh.at[pl.ds(rbase, RPT)])
    ebase = pl.multiple_of(wid * E_PER_W, 8)
    pltpu.sync_copy(srcf_hbm.at[pl.ds(ebase, E_PER_W)], src_v)
    pltpu.sync_copy(dstf_hbm.at[pl.ds(ebase, E_PER_W)], dst_v)
    plsc.subcore_barrier()
    obase = c * NP + rbase

    def start_gather(j, b):
        pltpu.async_copy(y_hbm.at[src_v.at[pl.ds(j * CHUNK, CHUNK)]], rows_v.at[b], gsem.at[b])

    def wait_gather(j, b):
        pltpu.make_async_copy(y_hbm.at[src_v.at[pl.ds(j * CHUNK, CHUNK)]], rows_v.at[b],
                              gsem.at[b]).wait()

    def start_scatter(j, b):
        pltpu.async_copy(rows_v.at[b], acc_sh.at[dst_v.at[pl.ds(j * CHUNK, CHUNK)]], ssem.at[b],
                         add=True)

    def wait_scatter(j, b):
        pltpu.make_async_copy(rows_v.at[b], acc_sh.at[dst_v.at[pl.ds(j * CHUNK, CHUNK)]],
                              ssem.at[b]).wait()

    for b in range(LAG):
        start_gather(b, b)

    def group(g, carry):
        for b in range(NBUF):
            v = g * NBUF + b
            jg = v + LAG
            bb = (b + LAG) % NBUF

            @pl.when(jg < NCHUNK)
            def _():
                @pl.when(jg >= NBUF)
                def _():
                    wait_scatter(jg - NBUF, bb)
                start_gather(jg, bb)

            wait_gather(v, b)
            start_scatter(v, b)
        return carry

    lax.fori_loop(0, NOUT, group, 0)
    for b in range(NBUF):
        wait_scatter(NCHUNK - NBUF + b, b)
    plsc.subcore_barrier()

    pltpu.sync_copy(acc_sh.at[pl.ds(rbase, RPT)],
                    acc_out.at[pl.ds(obase, RPT)])


def _dott(a, w):
    return lax.dot_general(a, w, (((1,), (1,)), ((), ())),
                           preferred_element_type=jnp.float32)


def _lin2_body(x_ref, wl_ref, wr_ref, y_ref, r_ref):
    xb = x_ref[...]
    xb = jnp.where(jnp.isnan(xb), jnp.float32(0), xb)
    y_ref[...] = _dott(xb, wl_ref[...]).astype(jnp.bfloat16)
    r_ref[...] = _dott(xb, wr_ref[...])


def _lin2(x, wl, wr):
    return pl.pallas_call(
        _lin2_body,
        grid=(NB,),
        in_specs=[
            pl.BlockSpec((BM, D_IN), lambda i: (i, 0)),
            pl.BlockSpec((H, D_IN), lambda i: (0, 0)),
            pl.BlockSpec((H, D_IN), lambda i: (0, 0)),
        ],
        out_specs=[
            pl.BlockSpec((BM, H), lambda i: (i, 0)),
            pl.BlockSpec((BM, H), lambda i: (i, 0)),
        ],
        out_shape=[
            jax.ShapeDtypeStruct((N, H), jnp.bfloat16),
            jax.ShapeDtypeStruct((N, H), jnp.float32),
        ],
    )(x, wl, wr)


def _norm_h(a0, a1, c0, c1, bl, r):
    cnt = jnp.maximum(c0[0] + c1[0], 1.0)
    agg = a0[0].astype(jnp.float32) + a1[0].astype(jnp.float32)
    return jnp.maximum(agg / cnt + bl + r, 0.0)


def _comb_body(a0_ref, a1_ref, c0_ref, c1_ref, r_ref, bl_ref, wl_ref, wr_ref,
               y_ref, rr_ref):
    h = _norm_h(a0_ref[...], a1_ref[...], c0_ref[...], c1_ref[...],
                bl_ref[...], r_ref[...])
    y_ref[...] = _dott(h, wl_ref[...]).astype(jnp.bfloat16)
    rr_ref[...] = _dott(h, wr_ref[...])


def _comb(acc, cnt, r, bl, wl, wr):
    return pl.pallas_call(
        _comb_body,
        grid=(NB,),
        in_specs=[
            pl.BlockSpec((1, BM, H), lambda i: (0, i, 0)),
            pl.BlockSpec((1, BM, H), lambda i: (1, i, 0)),
            pl.BlockSpec((1, BM, 1), lambda i: (0, i, 0)),
            pl.BlockSpec((1, BM, 1), lambda i: (1, i, 0)),
            pl.BlockSpec((BM, H), lambda i: (i, 0)),
            pl.BlockSpec((1, H), lambda i: (0, 0)),
            pl.BlockSpec((H, H), lambda i: (0, 0)),
            pl.BlockSpec((H, H), lambda i: (0, 0)),
        ],
        out_specs=[
            pl.BlockSpec((BM, H), lambda i: (i, 0)),
            pl.BlockSpec((BM, H), lambda i: (i, 0)),
        ],
        out_shape=[
            jax.ShapeDtypeStruct((N, H), jnp.bfloat16),
            jax.ShapeDtypeStruct((N, H), jnp.float32),
        ],
    )(acc, acc, cnt, cnt, r, bl, wl, wr)


def _final_body(a0_ref, a1_ref, c0_ref, c1_ref, r_ref, bl_ref, w_ref, bf_ref,
                o_ref):
    h = _norm_h(a0_ref[...], a1_ref[...], c0_ref[...], c1_ref[...],
                bl_ref[...], r_ref[...])
    o_ref[...] = (jnp.sum(h * w_ref[...], axis=1, keepdims=True)
                  + bf_ref[...])


def _final(acc, cnt, r, bl, w, bf):
    return pl.pallas_call(
        _final_body,
        grid=(NB,),
        in_specs=[
            pl.BlockSpec((1, BM, H), lambda i: (0, i, 0)),
            pl.BlockSpec((1, BM, H), lambda i: (1, i, 0)),
            pl.BlockSpec((1, BM, 1), lambda i: (0, i, 0)),
            pl.BlockSpec((1, BM, 1), lambda i: (1, i, 0)),
            pl.BlockSpec((BM, H), lambda i: (i, 0)),
            pl.BlockSpec((1, H), lambda i: (0, 0)),
            pl.BlockSpec((1, H), lambda i: (0, 0)),
            pl.BlockSpec((1, 1), lambda i: (0, 0)),
        ],
        out_specs=pl.BlockSpec((BM, 1), lambda i: (i, 0)),
        out_shape=jax.ShapeDtypeStruct((N, 1), jnp.float32),
    )(acc, acc, cnt, cnt, r, bl, w, bf)


def kernel(x, edge_index, Wl1, bl1, Wr1, Wl2, bl2, Wr2, Wfc, bfc):
    srcf = edge_index[0]
    dstf = edge_index[1]
    zr = jnp.zeros((RPT, H), jnp.bfloat16)
    zc = jnp.zeros((RPT,), jnp.float32)
    ones = jnp.ones((CHUNK,), jnp.float32)

    y1, r1 = _lin2(x, Wl1, Wr1)
    acc1, cnt = _seg_sum_count(y1, srcf, dstf, zr, zc, ones)
    acc1 = acc1.reshape(NSC, NP, H)
    cnt3 = cnt.reshape(NSC, NP, 1)
    y2, r2 = _comb(acc1, cnt3, r1, bl1.reshape(1, H), Wl2, Wr2)
    acc2 = _seg_sum(y2, srcf, dstf, zr).reshape(NSC, NP, H)
    out = _final(acc2, cnt3, r2, bl2.reshape(1, H), Wfc, bfc.reshape(1, 1))
    return out[:, 0]


# speedup vs baseline: 1.0794x; 1.0794x over previous
import functools

import jax
import jax.numpy as jnp
from jax import lax
from jax.experimental import pallas as pl
from jax.experimental.pallas import tpu as pltpu
from jax.experimental.pallas import tpu_sc as plsc

N = 10000
NP = 10240
E = 320000
D_IN = 128
H = 64
NSC = 2
NTILE = 16
NW = NSC * NTILE
E_PER_W = E // NW
CHUNK = 80
NCHUNK = E_PER_W // CHUNK
RPT = NP // NTILE
NBUF = 5
LAG = 4
NOUT = NCHUNK // NBUF

BM = 5000
NB = N // BM


def _mesh():
    return plsc.VectorSubcoreMesh(core_axis_name="c", subcore_axis_name="s")


@functools.partial(
    pl.kernel,
    out_type=(
        jax.ShapeDtypeStruct((NSC * NP, H), jnp.bfloat16),
        jax.ShapeDtypeStruct((NSC * NP,), jnp.float32),
    ),
    mesh=_mesh(),
    compiler_params=pltpu.CompilerParams(use_tc_tiling_on_sc=False),
    scratch_types=(
        pltpu.VMEM((NCHUNK, CHUNK), jnp.int32),
        pltpu.VMEM((NCHUNK, CHUNK), jnp.int32),
        pltpu.VMEM((NBUF, CHUNK, H), jnp.bfloat16),
        pltpu.VMEM((CHUNK,), jnp.float32),
        pltpu.VMEM_SHARED((NP, H), jnp.bfloat16),
        pltpu.VMEM_SHARED((NP,), jnp.float32),
        pltpu.SemaphoreType.DMA((NBUF,)),
        pltpu.SemaphoreType.DMA((NBUF,)),
        pltpu.SemaphoreType.DMA((NBUF,)),
    ),
)
def _seg_sum_count(y_hbm, ei_hbm, zr_hbm, zc_hbm, ones_hbm,
                   acc_out, cnt_out,
                   src_v, dst_v, rows_v, ones_v, acc_sh, cnt_sh,
                   gsem, ssem, csem):
    c = lax.axis_index("c")
    s = lax.axis_index("s")
    wid = c * NTILE + s
    rbase = s * RPT
    pltpu.sync_copy(zr_hbm, acc_sh.at[pl.ds(rbase, RPT)])
    pltpu.sync_copy(zc_hbm, cnt_sh.at[pl.ds(rbase, RPT)])
    pltpu.sync_copy(ei_hbm.at[0, wid], src_v)
    pltpu.sync_copy(ei_hbm.at[1, wid], dst_v)
    pltpu.sync_copy(ones_hbm, ones_v)
    plsc.subcore_barrier()
    obase = c * NP + rbase

    def start_gather(j, b):
        pltpu.async_copy(y_hbm.at[src_v.at[j]], rows_v.at[b], gsem.at[b])

    def wait_gather(j, b):
        pltpu.make_async_copy(y_hbm.at[src_v.at[j]], rows_v.at[b],
                              gsem.at[b]).wait()

    def start_scatter(j, b):
        pltpu.async_copy(rows_v.at[b], acc_sh.at[dst_v.at[j]], ssem.at[b],
                         add=True)

    def wait_scatter(j, b):
        pltpu.make_async_copy(rows_v.at[b], acc_sh.at[dst_v.at[j]],
                              ssem.at[b]).wait()

    def start_count(j, b):
        pltpu.async_copy(ones_v, cnt_sh.at[dst_v.at[j]], csem.at[b], add=True)

    def wait_count(j, b):
        pltpu.make_async_copy(ones_v, cnt_sh.at[dst_v.at[j]],
                              csem.at[b]).wait()

    for b in range(LAG):
        start_gather(b, b)

    def group(g, carry):
        for b in range(NBUF):
            v = g * NBUF + b
            jg = v + LAG
            bb = (b + LAG) % NBUF

            @pl.when(jg < NCHUNK)
            def _():
                @pl.when(jg >= NBUF)
                def _():
                    wait_scatter(jg - NBUF, bb)
                start_gather(jg, bb)

            wait_gather(v, b)
            start_scatter(v, b)

            @pl.when(v >= NBUF)
            def _():
                wait_count(v - NBUF, b)
            start_count(v, b)
        return carry

    lax.fori_loop(0, NOUT, group, 0)
    for b in range(NBUF):
        wait_scatter(NCHUNK - NBUF + b, b)
        wait_count(NCHUNK - NBUF + b, b)
    plsc.subcore_barrier()

    pltpu.sync_copy(acc_sh.at[pl.ds(rbase, RPT)],
                    acc_out.at[pl.ds(obase, RPT)])
    pltpu.sync_copy(cnt_sh.at[pl.ds(rbase, RPT)],
                    cnt_out.at[pl.ds(obase, RPT)])


@functools.partial(
    pl.kernel,
    out_type=jax.ShapeDtypeStruct((NSC * NP, H), jnp.bfloat16),
    mesh=_mesh(),
    compiler_params=pltpu.CompilerParams(use_tc_tiling_on_sc=False),
    scratch_types=(
        pltpu.VMEM((NCHUNK, CHUNK), jnp.int32),
        pltpu.VMEM((NCHUNK, CHUNK), jnp.int32),
        pltpu.VMEM((NBUF, CHUNK, H), jnp.bfloat16),
        pltpu.VMEM_SHARED((NP, H), jnp.bfloat16),
        pltpu.SemaphoreType.DMA((NBUF,)),
        pltpu.SemaphoreType.DMA((NBUF,)),
    ),
)
def _seg_sum(y_hbm, ei_hbm, zr_hbm,
             acc_out,
             src_v, dst_v, rows_v, acc_sh, gsem, ssem):
    c = lax.axis_index("c")
    s = lax.axis_index("s")
    wid = c * NTILE + s
    rbase = s * RPT
    pltpu.sync_copy(zr_hbm, acc_sh.at[pl.ds(rbase, RPT)])
    pltpu.sync_copy(ei_hbm.at[0, wid], src_v)
    pltpu.sync_copy(ei_hbm.at[1, wid], dst_v)
    plsc.subcore_barrier()
    obase = c * NP + rbase

    def start_gather(j, b):
        pltpu.async_copy(y_hbm.at[src_v.at[j]], rows_v.at[b], gsem.at[b])

    def wait_gather(j, b):
        pltpu.make_async_copy(y_hbm.at[src_v.at[j]], rows_v.at[b],
                              gsem.at[b]).wait()

    def start_scatter(j, b):
        pltpu.async_copy(rows_v.at[b], acc_sh.at[dst_v.at[j]], ssem.at[b],
                         add=True)

    def wait_scatter(j, b):
        pltpu.make_async_copy(rows_v.at[b], acc_sh.at[dst_v.at[j]],
                              ssem.at[b]).wait()

    for b in range(LAG):
        start_gather(b, b)

    def group(g, carry):
        for b in range(NBUF):
            v = g * NBUF + b
            jg = v + LAG
            bb = (b + LAG) % NBUF

            @pl.when(jg < NCHUNK)
            def _():
                @pl.when(jg >= NBUF)
                def _():
                    wait_scatter(jg - NBUF, bb)
                start_gather(jg, bb)

            wait_gather(v, b)
            start_scatter(v, b)
        return carry

    lax.fori_loop(0, NOUT, group, 0)
    for b in range(NBUF):
        wait_scatter(NCHUNK - NBUF + b, b)
    plsc.subcore_barrier()

    pltpu.sync_copy(acc_sh.at[pl.ds(rbase, RPT)],
                    acc_out.at[pl.ds(obase, RPT)])


def _dott(a, w):
    return lax.dot_general(a, w, (((1,), (1,)), ((), ())),
                           preferred_element_type=jnp.float32)


def _lin2_body(x_ref, wl_ref, wr_ref, y_ref, r_ref):
    xb = x_ref[...]
    xb = jnp.where(jnp.isnan(xb), jnp.float32(0), xb)
    y_ref[...] = _dott(xb, wl_ref[...]).astype(jnp.bfloat16)
    r_ref[...] = _dott(xb, wr_ref[...])


def _lin2(x, wl, wr):
    return pl.pallas_call(
        _lin2_body,
        grid=(NB,),
        in_specs=[
            pl.BlockSpec((BM, D_IN), lambda i: (i, 0)),
            pl.BlockSpec((H, D_IN), lambda i: (0, 0)),
            pl.BlockSpec((H, D_IN), lambda i: (0, 0)),
        ],
        out_specs=[
            pl.BlockSpec((BM, H), lambda i: (i, 0)),
            pl.BlockSpec((BM, H), lambda i: (i, 0)),
        ],
        out_shape=[
            jax.ShapeDtypeStruct((N, H), jnp.bfloat16),
            jax.ShapeDtypeStruct((N, H), jnp.float32),
        ],
    )(x, wl, wr)


def _norm_h(a0, a1, c0, c1, bl, r):
    cnt = jnp.maximum(c0[0] + c1[0], 1.0)
    agg = a0[0].astype(jnp.float32) + a1[0].astype(jnp.float32)
    return jnp.maximum(agg / cnt + bl + r, 0.0)


def _comb_body(a0_ref, a1_ref, c0_ref, c1_ref, r_ref, bl_ref, wl_ref, wr_ref,
               y_ref, rr_ref):
    h = _norm_h(a0_ref[...], a1_ref[...], c0_ref[...], c1_ref[...],
                bl_ref[...], r_ref[...])
    y_ref[...] = _dott(h, wl_ref[...]).astype(jnp.bfloat16)
    rr_ref[...] = _dott(h, wr_ref[...])


def _comb(acc, cnt, r, bl, wl, wr):
    return pl.pallas_call(
        _comb_body,
        grid=(NB,),
        in_specs=[
            pl.BlockSpec((1, BM, H), lambda i: (0, i, 0)),
            pl.BlockSpec((1, BM, H), lambda i: (1, i, 0)),
            pl.BlockSpec((1, BM, 1), lambda i: (0, i, 0)),
            pl.BlockSpec((1, BM, 1), lambda i: (1, i, 0)),
            pl.BlockSpec((BM, H), lambda i: (i, 0)),
            pl.BlockSpec((1, H), lambda i: (0, 0)),
            pl.BlockSpec((H, H), lambda i: (0, 0)),
            pl.BlockSpec((H, H), lambda i: (0, 0)),
        ],
        out_specs=[
            pl.BlockSpec((BM, H), lambda i: (i, 0)),
            pl.BlockSpec((BM, H), lambda i: (i, 0)),
        ],
        out_shape=[
            jax.ShapeDtypeStruct((N, H), jnp.bfloat16),
            jax.ShapeDtypeStruct((N, H), jnp.float32),
        ],
    )(acc, acc, cnt, cnt, r, bl, wl, wr)


def _final_body(a0_ref, a1_ref, c0_ref, c1_ref, r_ref, bl_ref, w_ref, bf_ref,
                o_ref):
    h = _norm_h(a0_ref[...], a1_ref[...], c0_ref[...], c1_ref[...],
                bl_ref[...], r_ref[...])
    o_ref[...] = (jnp.sum(h * w_ref[...], axis=1, keepdims=True)
                  + bf_ref[...])


def _final(acc, cnt, r, bl, w, bf):
    return pl.pallas_call(
        _final_body,
        grid=(NB,),
        in_specs=[
            pl.BlockSpec((1, BM, H), lambda i: (0, i, 0)),
            pl.BlockSpec((1, BM, H), lambda i: (1, i, 0)),
            pl.BlockSpec((1, BM, 1), lambda i: (0, i, 0)),
            pl.BlockSpec((1, BM, 1), lambda i: (1, i, 0)),
            pl.BlockSpec((BM, H), lambda i: (i, 0)),
            pl.BlockSpec((1, H), lambda i: (0, 0)),
            pl.BlockSpec((1, H), lambda i: (0, 0)),
            pl.BlockSpec((1, 1), lambda i: (0, 0)),
        ],
        out_specs=pl.BlockSpec((BM, 1), lambda i: (i, 0)),
        out_shape=jax.ShapeDtypeStruct((N, 1), jnp.float32),
    )(acc, acc, cnt, cnt, r, bl, w, bf)


def kernel(x, edge_index, Wl1, bl1, Wr1, Wl2, bl2, Wr2, Wfc, bfc):
    ei = edge_index.reshape(2, NW, NCHUNK, CHUNK)
    zr = jnp.zeros((RPT, H), jnp.bfloat16)
    zc = jnp.zeros((RPT,), jnp.float32)
    ones = jnp.ones((CHUNK,), jnp.float32)

    y1, r1 = _lin2(x, Wl1, Wr1)
    acc1, cnt = _seg_sum_count(y1, ei, zr, zc, ones)
    acc1 = acc1.reshape(NSC, NP, H)
    cnt3 = cnt.reshape(NSC, NP, 1)
    y2, r2 = _comb(acc1, cnt3, r1, bl1.reshape(1, H), Wl2, Wr2)
    acc2 = _seg_sum(y2, ei, zr).reshape(NSC, NP, H)
    out = _final(acc2, cnt3, r2, bl2.reshape(1, H), Wfc, bfc.reshape(1, 1))
    return out[:, 0]
